# baseline (speedup 1.0000x reference)
import functools

import jax
import jax.numpy as jnp
from jax import lax
from jax.experimental import pallas as pl
from jax.experimental.pallas import tpu as pltpu
from jax.experimental.pallas import tpu_sc as plsc

_VOCAB = 100000
_EMBED = 64
_BATCH = 4096

_TV = 2048
_NV = (_VOCAB + _TV - 1) // _TV
_VPAD = _NV * _TV
_TQ = 1024
_NQ = _BATCH // _TQ
_KDIM = _EMBED + 1
_DPAD = 128


@functools.cache
def _make_sc_gather():
    info = plsc.get_sparse_core_info()
    nc, ns = info.num_cores, info.num_subcores
    nw = nc * ns
    b_per_w = _BATCH // nw

    mesh = plsc.VectorSubcoreMesh(core_axis_name="c", subcore_axis_name="s")

    @functools.partial(
        pl.kernel,
        mesh=mesh,
        out_type=jax.ShapeDtypeStruct((_BATCH, _DPAD), jnp.float32),
        scratch_types=[
            pltpu.VMEM((b_per_w,), jnp.int32),
            pltpu.VMEM((b_per_w, _DPAD), jnp.float32),
            pltpu.SemaphoreType.DMA,
        ],
    )
    def gather(idx_hbm, table_hbm, out_hbm, idx_v, rows_v, sem):
        wid = lax.axis_index("s") * nc + lax.axis_index("c")
        base = wid * b_per_w
        pltpu.sync_copy(idx_hbm.at[pl.ds(base, b_per_w)], idx_v)
        pltpu.async_copy(table_hbm.at[idx_v], rows_v, sem).wait()
        pltpu.sync_copy(rows_v, out_hbm.at[pl.ds(base, b_per_w)])

    return gather


def _fused_body(w_ref, emb_ref, o_ref, s_ref, lse_ref):
    p = pl.program_id(0)
    j = pl.program_id(1)

    def tile_dot(e_half):
        return lax.dot_general(
            w_ref[:], e_half, (((1,), (1,)), ((), ())),
            preferred_element_type=jnp.float32,
        )

    def expfold(lt):
        e = jnp.exp(lt)
        acc = e[0:128, :]
        for k in range(1, _TV // 128):
            acc = acc + e[k * 128:(k + 1) * 128, :]
        return acc

    def eq(q):
        return emb_ref[q * _TQ:(q + 1) * _TQ, :]

    def scol(q):
        return slice(q * _TQ, (q + 1) * _TQ)

    for q in range(_NQ):
        @pl.when(jnp.logical_and(p == q, j == 0))
        def _(q=q):
            s_ref[:, scol(q)] = jnp.zeros_like(s_ref[:, scol(q)])

        @pl.when(p == q)
        def _(q=q):
            s_ref[:, scol(q)] += expfold(tile_dot(eq(q)))

    for q in range(_NQ):
        @pl.when(jnp.logical_and(p == q + 1, j == 0))
        def _(q=q):
            lse_ref[:, scol(q)] = jnp.log(
                jnp.sum(s_ref[:, scol(q)], axis=0, keepdims=True))

        @pl.when(p == q + 1)
        def _(q=q):
            o_ref[:] = tile_dot(eq(q)) - lse_ref[:, scol(q)]


def _log_probs_fused(embeds, wb):
    return pl.pallas_call(
        _fused_body,
        grid=(_NQ + 1, _NV),
        in_specs=[
            pl.BlockSpec((_TV, _KDIM), lambda p, j: (j, 0)),
            pl.BlockSpec((_BATCH, _KDIM), lambda p, j: (0, 0)),
        ],
        out_specs=pl.BlockSpec(
            (_TV, _TQ),
            lambda p, j: (jnp.where(p == 0, 0, j),
                          jnp.where(p == 0, 0, p - 1)),
        ),
        out_shape=jax.ShapeDtypeStruct((_VOCAB, _BATCH), jnp.float32),
        scratch_shapes=[
            pltpu.VMEM((128, _BATCH), jnp.float32),
            pltpu.VMEM((1, _BATCH), jnp.float32),
        ],
        compiler_params=pltpu.CompilerParams(
            dimension_semantics=("arbitrary", "arbitrary"),
        ),
    )(wb, embeds)


def kernel(inputs, emb_table, W, b):
    idx = inputs.astype(jnp.int32)
    table128 = jnp.concatenate(
        [emb_table,
         jnp.ones((_VOCAB, 1), jnp.float32),
         jnp.zeros((_VOCAB, _DPAD - _EMBED - 1), jnp.float32)], axis=1)
    embeds = _make_sc_gather()(idx, table128)[:, :_KDIM]
    wb = jnp.concatenate([W, b[:, None]], axis=1)
    pad = jnp.concatenate(
        [jnp.zeros((_VPAD - _VOCAB, _EMBED), jnp.float32),
         jnp.full((_VPAD - _VOCAB, 1), -1e30, jnp.float32)], axis=1)
    wb = jnp.concatenate([wb, pad], axis=0)
    return _log_probs_fused(embeds, wb).T


# speedup vs baseline: 1.0072x; 1.0072x over previous
import functools

import jax
import jax.numpy as jnp
from jax import lax
from jax.experimental import pallas as pl
from jax.experimental.pallas import tpu as pltpu
from jax.experimental.pallas import tpu_sc as plsc

_VOCAB = 100000
_EMBED = 64
_BATCH = 4096

_TV = 2048
_NV = (_VOCAB + _TV - 1) // _TV
_VPAD = _NV * _TV
_TB = 2048
_KDIM = _EMBED + 1
_DPAD = 128


@functools.cache
def _make_sc_gather():
    info = plsc.get_sparse_core_info()
    nc, ns = info.num_cores, info.num_subcores
    nw = nc * ns
    b_per_w = _BATCH // nw

    mesh = plsc.VectorSubcoreMesh(core_axis_name="c", subcore_axis_name="s")

    @functools.partial(
        pl.kernel,
        mesh=mesh,
        out_type=jax.ShapeDtypeStruct((_BATCH, _DPAD), jnp.float32),
        scratch_types=[
            pltpu.VMEM((b_per_w,), jnp.int32),
            pltpu.VMEM((b_per_w, _DPAD), jnp.float32),
            pltpu.SemaphoreType.DMA,
        ],
    )
    def gather(idx_hbm, table_hbm, out_hbm, idx_v, rows_v, sem):
        wid = lax.axis_index("s") * nc + lax.axis_index("c")
        base = wid * b_per_w
        pltpu.sync_copy(idx_hbm.at[pl.ds(base, b_per_w)], idx_v)
        pltpu.async_copy(table_hbm.at[idx_v], rows_v, sem).wait()
        pltpu.sync_copy(rows_v, out_hbm.at[pl.ds(base, b_per_w)])

    return gather


def _fused_body(w_ref, emb_ref, o_ref, s_ref, lse_ref):
    p = pl.program_id(0)
    j = pl.program_id(1)

    def tile_dot(e_half):
        return lax.dot_general(
            w_ref[:], e_half, (((1,), (1,)), ((), ())),
            preferred_element_type=jnp.float32,
        )

    def expfold(lt):
        e = jnp.exp(lt)
        acc = e[0:128, :]
        for k in range(1, _TV // 128):
            acc = acc + e[k * 128:(k + 1) * 128, :]
        return acc

    e0 = emb_ref[0:_TB, :]
    e1 = emb_ref[_TB:, :]

    @pl.when(jnp.logical_and(p == 0, j == 0))
    def _():
        s_ref[:, 0:_TB] = jnp.zeros_like(s_ref[:, 0:_TB])

    @pl.when(p == 0)
    def _():
        s_ref[:, 0:_TB] += expfold(tile_dot(e0))

    @pl.when(jnp.logical_and(p == 1, j == 0))
    def _():
        lse_ref[:, 0:_TB] = jnp.log(
            jnp.sum(s_ref[:, 0:_TB], axis=0, keepdims=True))
        s_ref[:, _TB:] = jnp.zeros_like(s_ref[:, _TB:])

    @pl.when(p == 1)
    def _():
        s_ref[:, _TB:] += expfold(tile_dot(e1))
        o_ref[:] = tile_dot(e0) - lse_ref[:, 0:_TB]

    @pl.when(jnp.logical_and(p == 2, j == 0))
    def _():
        lse_ref[:, _TB:] = jnp.log(
            jnp.sum(s_ref[:, _TB:], axis=0, keepdims=True))

    @pl.when(p == 2)
    def _():
        o_ref[:] = tile_dot(e1) - lse_ref[:, _TB:]


def _log_probs_fused(embeds, wb):
    return pl.pallas_call(
        _fused_body,
        grid=(3, _NV),
        in_specs=[
            pl.BlockSpec((_TV, _KDIM), lambda p, j: (j, 0)),
            pl.BlockSpec((_BATCH, _KDIM), lambda p, j: (0, 0)),
        ],
        out_specs=pl.BlockSpec(
            (_TV, _TB),
            lambda p, j: (jnp.where(p == 0, 0, j), jnp.where(p == 2, 1, 0)),
        ),
        out_shape=jax.ShapeDtypeStruct((_VOCAB, _BATCH), jnp.float32),
        scratch_shapes=[
            pltpu.VMEM((128, _BATCH), jnp.float32),
            pltpu.VMEM((1, _BATCH), jnp.float32),
        ],
        compiler_params=pltpu.CompilerParams(
            dimension_semantics=("arbitrary", "arbitrary"),
        ),
    )(wb, embeds)


def kernel(inputs, emb_table, W, b):
    idx = inputs.astype(jnp.int32)
    table128 = jnp.concatenate(
        [emb_table,
         jnp.ones((_VOCAB, 1), jnp.float32),
         jnp.zeros((_VOCAB, _DPAD - _EMBED - 1), jnp.float32)], axis=1)
    embeds = _make_sc_gather()(idx, table128)[:, :_KDIM]
    wb = jnp.concatenate([W, b[:, None]], axis=1)
    pad = jnp.concatenate(
        [jnp.zeros((_VPAD - _VOCAB, _EMBED), jnp.float32),
         jnp.full((_VPAD - _VOCAB, 1), -1e30, jnp.float32)], axis=1)
    wb = jnp.concatenate([wb, pad], axis=0)
    return _log_probs_fused(embeds.astype(jnp.bfloat16),
                            wb.astype(jnp.bfloat16)).T


# speedup vs baseline: 1.0209x; 1.0135x over previous
import functools

import jax
import jax.numpy as jnp
from jax import lax
from jax.experimental import pallas as pl
from jax.experimental.pallas import tpu as pltpu
from jax.experimental.pallas import tpu_sc as plsc

_VOCAB = 100000
_EMBED = 64
_BATCH = 4096

_TV = 2048
_NV = (_VOCAB + _TV - 1) // _TV
_VPAD = _NV * _TV
_TB = 2048
_KDIM = _EMBED + 1
_DPAD = 128


@functools.cache
def _make_sc_gather():
    info = plsc.get_sparse_core_info()
    nc, ns = info.num_cores, info.num_subcores
    nw = nc * ns
    b_per_w = _BATCH // nw

    mesh = plsc.VectorSubcoreMesh(core_axis_name="c", subcore_axis_name="s")

    @functools.partial(
        pl.kernel,
        mesh=mesh,
        out_type=jax.ShapeDtypeStruct((_BATCH, _DPAD), jnp.float32),
        scratch_types=[
            pltpu.VMEM((b_per_w,), jnp.int32),
            pltpu.VMEM((b_per_w, _DPAD), jnp.float32),
            pltpu.SemaphoreType.DMA,
        ],
    )
    def gather(idx_hbm, table_hbm, out_hbm, idx_v, rows_v, sem):
        wid = lax.axis_index("s") * nc + lax.axis_index("c")
        base = wid * b_per_w
        pltpu.sync_copy(idx_hbm.at[pl.ds(base, b_per_w)], idx_v)
        pltpu.async_copy(table_hbm.at[idx_v], rows_v, sem).wait()
        pltpu.sync_copy(rows_v, out_hbm.at[pl.ds(base, b_per_w)])

    return gather


def _fused_body(w_ref, emb_ref, o_ref, s_ref, lse_ref):
    p = pl.program_id(0)
    j = pl.program_id(1)

    def tile_dot(e_half):
        return lax.dot_general(
            w_ref[:], e_half, (((1,), (1,)), ((), ())),
            preferred_element_type=jnp.float32,
        )

    def expfold(lt):
        e = jnp.exp(lt)
        acc = e[0:128, :]
        for k in range(1, _TV // 128):
            acc = acc + e[k * 128:(k + 1) * 128, :]
        return acc

    e0 = emb_ref[0:_TB, :]
    e1 = emb_ref[_TB:, :]

    @pl.when(jnp.logical_and(p == 0, j == 0))
    def _():
        s_ref[:, 0:_TB] = jnp.zeros_like(s_ref[:, 0:_TB])

    @pl.when(p == 0)
    def _():
        s_ref[:, 0:_TB] += expfold(tile_dot(e0))

    @pl.when(jnp.logical_and(p == 1, j == 0))
    def _():
        lse_ref[:, 0:_TB] = jnp.log(
            jnp.sum(s_ref[:, 0:_TB], axis=0, keepdims=True))
        s_ref[:, _TB:] = jnp.zeros_like(s_ref[:, _TB:])

    @pl.when(p == 1)
    def _():
        s_ref[:, _TB:] += expfold(tile_dot(e1))
        o_ref[:] = tile_dot(e0) - lse_ref[:, 0:_TB]

    @pl.when(jnp.logical_and(p == 2, j == 0))
    def _():
        lse_ref[:, _TB:] = jnp.log(
            jnp.sum(s_ref[:, _TB:], axis=0, keepdims=True))

    @pl.when(p == 2)
    def _():
        o_ref[:] = tile_dot(e1) - lse_ref[:, _TB:]


def _log_probs_fused(embeds, wb):
    return pl.pallas_call(
        _fused_body,
        grid=(3, _NV),
        in_specs=[
            pl.BlockSpec((_TV, _KDIM), lambda p, j: (j, 0)),
            pl.BlockSpec((_BATCH, _KDIM), lambda p, j: (0, 0)),
        ],
        out_specs=pl.BlockSpec(
            (_TV, _TB),
            lambda p, j: (jnp.where(p == 0, 0, j), jnp.where(p == 2, 1, 0)),
        ),
        out_shape=jax.ShapeDtypeStruct((_VOCAB, _BATCH), jnp.float32),
        scratch_shapes=[
            pltpu.VMEM((128, _BATCH), jnp.float32),
            pltpu.VMEM((1, _BATCH), jnp.float32),
        ],
        compiler_params=pltpu.CompilerParams(
            dimension_semantics=("arbitrary", "arbitrary"),
        ),
    )(wb, embeds)


def kernel(inputs, emb_table, W, b):
    idx = inputs.astype(jnp.int32)
    table128 = jnp.concatenate(
        [emb_table,
         jnp.ones((_VOCAB, 1), jnp.float32),
         jnp.zeros((_VOCAB, _DPAD - _EMBED - 1), jnp.float32)], axis=1)
    embeds = _make_sc_gather()(idx, table128)[:, :_KDIM]
    wb = jnp.concatenate([W, b[:, None]], axis=1)
    pad = jnp.concatenate(
        [jnp.zeros((_VPAD - _VOCAB, _EMBED), jnp.float32),
         jnp.full((_VPAD - _VOCAB, 1), -1e30, jnp.float32)], axis=1)
    wb = jnp.concatenate([wb, pad], axis=0)
    return _log_probs_fused(embeds, wb).T
